# baseline (speedup 1.0000x reference)
import jax
import jax.numpy as jnp
from jax.experimental import pallas as pl
from jax.experimental.pallas import tpu as pltpu


def _cov_kernel(p_ref, t_ref, out_ref):
    out_ref[...] = (p_ref[...] + t_ref[...]).sum()[None, None]


def kernel(predict_probs, true_labels):
    out = pl.pallas_call(
        _cov_kernel,
        grid=(1,),
        in_specs=[
            pl.BlockSpec((8, 1000), lambda i: (0, 0)),
            pl.BlockSpec((8, 1000), lambda i: (0, 0)),
        ],
        out_specs=pl.BlockSpec((1, 1), lambda i: (0, 0)),
        out_shape=jax.ShapeDtypeStruct((1, 1), jnp.float32),
    )(predict_probs, true_labels)
    return out[0, 0] / 4096.0


# speedup vs baseline: 2.4997x; 2.4997x over previous
import jax
import jax.numpy as jnp
from jax.experimental import pallas as pl

N_ROWS = 4096
N_COLS = 1000
BC = 512


def _cov_kernel(p_ref, t_ref, out_ref):
    p = p_ref[...]
    t = t_ref[...]
    masked = jnp.where(t > 0, p, jnp.inf)
    colmin = jnp.min(masked, axis=0, keepdims=True)
    cov = jnp.sum((p >= colmin).astype(jnp.float32), axis=0)
    cov = jnp.where(jnp.isfinite(colmin[0, :]), cov, 0.0)
    total = jnp.sum(cov)

    @pl.when(pl.program_id(0) == 0)
    def _():
        out_ref[...] = jnp.zeros((1, 1), jnp.float32)

    out_ref[...] += total[None, None]


def kernel(predict_probs, true_labels):
    p = predict_probs.T
    t = true_labels.T
    out = pl.pallas_call(
        _cov_kernel,
        grid=(N_ROWS // BC,),
        in_specs=[
            pl.BlockSpec((N_COLS, BC), lambda i: (0, i)),
            pl.BlockSpec((N_COLS, BC), lambda i: (0, i)),
        ],
        out_specs=pl.BlockSpec((1, 1), lambda i: (0, 0)),
        out_shape=jax.ShapeDtypeStruct((1, 1), jnp.float32),
    )(p, t)
    return out[0, 0] / N_ROWS


# speedup vs baseline: 2.7056x; 1.0824x over previous
import jax
import jax.numpy as jnp
from jax.experimental import pallas as pl

N_ROWS = 4096
N_COLS = 1000
BC = 1024


def _cov_kernel(p_ref, t_ref, out_ref):
    p = p_ref[...]
    t = t_ref[...]
    masked = jnp.where(t > 0, p, jnp.inf)
    colmin = jnp.min(masked, axis=0, keepdims=True)
    cov = jnp.sum((p >= colmin).astype(jnp.float32), axis=0)
    cov = jnp.where(jnp.isfinite(colmin[0, :]), cov, 0.0)
    total = jnp.sum(cov)

    @pl.when(pl.program_id(0) == 0)
    def _():
        out_ref[...] = jnp.zeros((1, 1), jnp.float32)

    out_ref[...] += total[None, None]


def kernel(predict_probs, true_labels):
    p = predict_probs.T
    t = true_labels.T
    out = pl.pallas_call(
        _cov_kernel,
        grid=(N_ROWS // BC,),
        in_specs=[
            pl.BlockSpec((N_COLS, BC), lambda i: (0, i)),
            pl.BlockSpec((N_COLS, BC), lambda i: (0, i)),
        ],
        out_specs=pl.BlockSpec((1, 1), lambda i: (0, 0)),
        out_shape=jax.ShapeDtypeStruct((1, 1), jnp.float32),
    )(p, t)
    return out[0, 0] / N_ROWS
